# Initial kernel scaffold; baseline (speedup 1.0000x reference)
import jax
import jax.numpy as jnp
from jax.experimental import pallas as pl


def kernel(x, W):
    raise NotImplementedError("write your pallas kernel here")


# speedup vs baseline: 1.0583x; 1.0583x over previous
import functools

import jax
import jax.numpy as jnp
from jax.experimental import pallas as pl
from jax.experimental.pallas import tpu as pltpu

_TOP_K = 8


def _router_body(x_ref, wt_ref, probs_ref, w_ref, i_ref):
    logits = jnp.dot(x_ref[...], wt_ref[...], preferred_element_type=jnp.float32)
    m = jnp.max(logits, axis=-1, keepdims=True)
    e = jnp.exp(logits - m)
    s = jnp.sum(e, axis=-1, keepdims=True)
    probs = e / s
    probs_ref[...] = probs

    n_exp = probs.shape[-1]
    lane = jax.lax.broadcasted_iota(jnp.int32, probs.shape, dimension=1)
    work = probs
    ws = []
    idxs = []
    for _ in range(_TOP_K):
        mx = jnp.max(work, axis=-1, keepdims=True)
        is_max = work == mx
        cand = jnp.where(is_max, lane, n_exp)
        sel = jnp.min(cand, axis=-1, keepdims=True)
        ws.append(mx)
        idxs.append(sel)
        work = jnp.where(lane == sel, -1.0, work)
    w = jnp.concatenate(ws, axis=1)
    idx = jnp.concatenate(idxs, axis=1)
    w = w / jnp.sum(w, axis=-1, keepdims=True)
    w_ref[...] = w
    i_ref[...] = idx


def kernel(x, W):
    b, s, h = x.shape
    n_exp = W.shape[0]
    n = b * s
    xf = x.reshape(n, h)
    wt = W.T

    tile = 512
    while n % tile:
        tile //= 2
    grid = (n // tile,)

    probs, w, idx = pl.pallas_call(
        _router_body,
        grid=grid,
        in_specs=[
            pl.BlockSpec((tile, h), lambda i: (i, 0)),
            pl.BlockSpec((h, n_exp), lambda i: (0, 0)),
        ],
        out_specs=[
            pl.BlockSpec((tile, n_exp), lambda i: (i, 0)),
            pl.BlockSpec((tile, _TOP_K), lambda i: (i, 0)),
            pl.BlockSpec((tile, _TOP_K), lambda i: (i, 0)),
        ],
        out_shape=[
            jax.ShapeDtypeStruct((n, n_exp), jnp.float32),
            jax.ShapeDtypeStruct((n, _TOP_K), jnp.float32),
            jax.ShapeDtypeStruct((n, _TOP_K), jnp.int32),
        ],
    )(xf, wt)

    return (
        w.reshape(b, s, _TOP_K),
        idx.reshape(b, s, _TOP_K),
        probs.reshape(b, s, n_exp),
    )


# speedup vs baseline: 1.1919x; 1.1262x over previous
import functools

import jax
import jax.numpy as jnp
from jax.experimental import pallas as pl
from jax.experimental.pallas import tpu as pltpu

_TOP_K = 8


def _router_body(x_ref, wt_ref, probs_ref, w_ref, i_ref):
    logits = jnp.dot(x_ref[...], wt_ref[...], preferred_element_type=jnp.float32)
    m = jnp.max(logits, axis=-1, keepdims=True)
    e = jnp.exp(logits - m)
    s = jnp.sum(e, axis=-1, keepdims=True)
    probs = e / s
    probs_ref[...] = probs

    n_exp = probs.shape[-1]
    lane = jax.lax.broadcasted_iota(jnp.int32, probs.shape, dimension=1)
    work = probs
    ws = []
    idxs = []
    for _ in range(_TOP_K):
        mx = jnp.max(work, axis=-1, keepdims=True)
        is_max = work == mx
        cand = jnp.where(is_max, lane, n_exp)
        sel = jnp.min(cand, axis=-1, keepdims=True)
        ws.append(mx)
        idxs.append(sel)
        work = jnp.where(lane == sel, -1.0, work)
    w = jnp.concatenate(ws, axis=1)
    idx = jnp.concatenate(idxs, axis=1)
    w = w / jnp.sum(w, axis=-1, keepdims=True)
    w_ref[...] = w
    i_ref[...] = idx


def kernel(x, W):
    b, s, h = x.shape
    n_exp = W.shape[0]
    n = b * s
    xf = x.reshape(n, h)
    wt = W.T

    tile = 1024
    while n % tile:
        tile //= 2
    grid = (n // tile,)

    probs, w, idx = pl.pallas_call(
        _router_body,
        grid=grid,
        in_specs=[
            pl.BlockSpec((tile, h), lambda i: (i, 0)),
            pl.BlockSpec((h, n_exp), lambda i: (0, 0)),
        ],
        out_specs=[
            pl.BlockSpec((tile, n_exp), lambda i: (i, 0)),
            pl.BlockSpec((tile, _TOP_K), lambda i: (i, 0)),
            pl.BlockSpec((tile, _TOP_K), lambda i: (i, 0)),
        ],
        out_shape=[
            jax.ShapeDtypeStruct((n, n_exp), jnp.float32),
            jax.ShapeDtypeStruct((n, _TOP_K), jnp.float32),
            jax.ShapeDtypeStruct((n, _TOP_K), jnp.int32),
        ],
    )(xf, wt)

    return (
        w.reshape(b, s, _TOP_K),
        idx.reshape(b, s, _TOP_K),
        probs.reshape(b, s, n_exp),
    )
